# baseline (speedup 1.0000x reference)
import functools

import jax
import jax.numpy as jnp
from jax import lax
from jax.experimental import pallas as pl
from jax.experimental.pallas import tpu as pltpu
from jax.experimental.pallas import tpu_sc as plsc

_TOP_K = 8
_NUM_EXPERTS = 64
_ROW_BLOCK = 512
_LANES = 16


def _logits_block(hs_ref, w_ref, b_ref, out_ref):
    out_ref[...] = (
        jnp.dot(hs_ref[...], w_ref[...], preferred_element_type=jnp.float32)
        + b_ref[...]
    )


def _tc_logits(hs, w, bias2d):
    n_rows, hidden_dim = hs.shape
    grid = (n_rows // _ROW_BLOCK,)
    return pl.pallas_call(
        _logits_block,
        grid=grid,
        in_specs=[
            pl.BlockSpec((_ROW_BLOCK, hidden_dim), lambda i: (i, 0)),
            pl.BlockSpec((hidden_dim, _NUM_EXPERTS), lambda i: (0, 0)),
            pl.BlockSpec((1, _NUM_EXPERTS), lambda i: (0, 0)),
        ],
        out_specs=pl.BlockSpec((_ROW_BLOCK, _NUM_EXPERTS), lambda i: (i, 0)),
        out_shape=jax.ShapeDtypeStruct((n_rows, _NUM_EXPERTS), jnp.float32),
        compiler_params=pltpu.CompilerParams(
            dimension_semantics=("arbitrary",),
        ),
    )(hs, w, bias2d)


_GATHER_DNUMS = lax.GatherDimensionNumbers(
    offset_dims=(), collapsed_slice_dims=(0,), start_index_map=(0,)
)


def _shuffle(x, idx):
    return lax.gather(
        x,
        idx[:, None],
        _GATHER_DNUMS,
        (1,),
        mode=lax.GatherScatterMode.PROMISE_IN_BOUNDS,
    )


def _butterfly(x, op, perms):
    for p in perms:
        x = op(x, _shuffle(x, p))
    return x


def _process_row(logits_v, scores_v, r, idx_vecs, lane_iota, perms, lane_off, acc):
    neg_inf = jnp.float32(-jnp.inf)
    v = [logits_v[r, pl.ds(16 * q, 16)] for q in range(4)]
    w = list(v)
    rowmax = None
    for k in range(_TOP_K):
        t = jnp.maximum(jnp.maximum(w[0], w[1]), jnp.maximum(w[2], w[3]))
        m = _butterfly(t, jnp.maximum, perms)
        if k == 0:
            rowmax = m
        cand = [
            jnp.where(w[q] == m, idx_vecs[q], jnp.int32(_NUM_EXPERTS))
            for q in range(4)
        ]
        cmin = jnp.minimum(
            jnp.minimum(cand[0], cand[1]), jnp.minimum(cand[2], cand[3])
        )
        chosen = _butterfly(cmin, jnp.minimum, perms)
        acc = jnp.where(lane_iota == jnp.int32(lane_off + k), chosen, acc)
        for q in range(4):
            w[q] = jnp.where(idx_vecs[q] == chosen, neg_inf, w[q])
    e = [
        jnp.where(w[q] == neg_inf, jnp.exp(v[q] - rowmax), jnp.float32(0.0))
        for q in range(4)
    ]
    denom = _butterfly(e[0] + e[1] + e[2] + e[3], jnp.add, perms)
    inv = 1.0 / denom
    for q in range(4):
        scores_v[r, pl.ds(16 * q, 16)] = e[q] * inv
    return acc


def _sc_router(logits):
    n_rows = logits.shape[0]
    nc, ns = 2, 16
    nw = nc * ns
    rows_per_w = n_rows // nw
    pairs_per_w = rows_per_w // 2
    idx_rows = n_rows // 2

    mesh = plsc.VectorSubcoreMesh(core_axis_name="c", subcore_axis_name="s")

    @functools.partial(
        pl.kernel,
        out_type=[
            jax.ShapeDtypeStruct((n_rows, _NUM_EXPERTS), jnp.float32),
            jax.ShapeDtypeStruct((idx_rows, _LANES), jnp.int32),
        ],
        mesh=mesh,
        scratch_types=[
            pltpu.VMEM((rows_per_w, _NUM_EXPERTS), jnp.float32),
            pltpu.VMEM((rows_per_w, _NUM_EXPERTS), jnp.float32),
            pltpu.VMEM((pairs_per_w, _LANES), jnp.int32),
        ],
    )
    def sc_kernel(logits_hbm, scores_hbm, idx_hbm, logits_v, scores_v, idx_v):
        wid = lax.axis_index("s") * nc + lax.axis_index("c")
        base = wid * rows_per_w
        pltpu.sync_copy(logits_hbm.at[pl.ds(base, rows_per_w)], logits_v)

        lane_iota = lax.iota(jnp.int32, _LANES)
        idx_vecs = [lane_iota + jnp.int32(16 * q) for q in range(4)]
        perms = [jnp.bitwise_xor(lane_iota, jnp.int32(b)) for b in (1, 2, 4, 8)]

        @plsc.parallel_loop(0, pairs_per_w, step=1, unroll=4)
        def pair_body(p):
            acc = jnp.zeros((_LANES,), dtype=jnp.int32)
            acc = _process_row(
                logits_v, scores_v, 2 * p, idx_vecs, lane_iota, perms, 0, acc
            )
            acc = _process_row(
                logits_v, scores_v, 2 * p + 1, idx_vecs, lane_iota, perms, 8, acc
            )
            idx_v[p, ...] = acc

        pltpu.sync_copy(scores_v, scores_hbm.at[pl.ds(base, rows_per_w)])
        pltpu.sync_copy(idx_v, idx_hbm.at[pl.ds(wid * pairs_per_w, pairs_per_w)])

    return sc_kernel(logits)


def kernel(hidden_states, kernel, bias):
    hidden_dim = hidden_states.shape[-1]
    hs = hidden_states.reshape(-1, hidden_dim)
    n_rows = hs.shape[0]
    bias2d = bias.reshape(1, _NUM_EXPERTS)
    logits = _tc_logits(hs, kernel, bias2d)
    scores, idx_packed = _sc_router(logits)
    return scores, idx_packed.reshape(n_rows, _TOP_K)


# speedup vs baseline: 1.2713x; 1.2713x over previous
import functools

import jax
import jax.numpy as jnp
from jax import lax
from jax.experimental import pallas as pl
from jax.experimental.pallas import tpu as pltpu
from jax.experimental.pallas import tpu_sc as plsc

_TOP_K = 8
_NUM_EXPERTS = 64
_ROW_BLOCK = 512
_LANES = 16


def _logits_block(hs_ref, w_ref, b_ref, out_ref):
    out_ref[...] = (
        jnp.dot(hs_ref[...], w_ref[...], preferred_element_type=jnp.float32)
        + b_ref[...]
    )


def _tc_logits(hs, w, bias2d):
    n_rows, hidden_dim = hs.shape
    grid = (n_rows // _ROW_BLOCK,)
    return pl.pallas_call(
        _logits_block,
        grid=grid,
        in_specs=[
            pl.BlockSpec((_ROW_BLOCK, hidden_dim), lambda i: (i, 0)),
            pl.BlockSpec((hidden_dim, _NUM_EXPERTS), lambda i: (0, 0)),
            pl.BlockSpec((1, _NUM_EXPERTS), lambda i: (0, 0)),
        ],
        out_specs=pl.BlockSpec((_ROW_BLOCK, _NUM_EXPERTS), lambda i: (i, 0)),
        out_shape=jax.ShapeDtypeStruct((n_rows, _NUM_EXPERTS), jnp.float32),
        compiler_params=pltpu.CompilerParams(
            dimension_semantics=("arbitrary",),
        ),
    )(hs, w, bias2d)


_GATHER_DNUMS = lax.GatherDimensionNumbers(
    offset_dims=(), collapsed_slice_dims=(0,), start_index_map=(0,)
)


def _shuffle(x, idx):
    return lax.gather(
        x,
        idx[:, None],
        _GATHER_DNUMS,
        (1,),
        mode=lax.GatherScatterMode.PROMISE_IN_BOUNDS,
    )


def _butterfly(x, op, perms):
    for p in perms:
        x = op(x, _shuffle(x, p))
    return x


def _process_row(logits_v, scores_v, r, idx_vecs, lane_iota, perms, lane_off, acc):
    neg_inf = jnp.float32(-jnp.inf)
    v = [logits_v[r, pl.ds(16 * q, 16)] for q in range(4)]
    w = list(v)
    rowmax = None
    for k in range(_TOP_K):
        t = jnp.maximum(jnp.maximum(w[0], w[1]), jnp.maximum(w[2], w[3]))
        m = _butterfly(t, jnp.maximum, perms)
        if k == 0:
            rowmax = m
        cand = [
            jnp.where(w[q] == m, idx_vecs[q], jnp.int32(_NUM_EXPERTS))
            for q in range(4)
        ]
        cmin = jnp.minimum(
            jnp.minimum(cand[0], cand[1]), jnp.minimum(cand[2], cand[3])
        )
        chosen = _butterfly(cmin, jnp.minimum, perms)
        acc = jnp.where(lane_iota == jnp.int32(lane_off + k), chosen, acc)
        for q in range(4):
            w[q] = jnp.where(idx_vecs[q] == chosen, neg_inf, w[q])
    e = [
        jnp.where(w[q] == neg_inf, jnp.exp(v[q] - rowmax), jnp.float32(0.0))
        for q in range(4)
    ]
    denom = _butterfly(e[0] + e[1] + e[2] + e[3], jnp.add, perms)
    inv = 1.0 / denom
    for q in range(4):
        scores_v[r, pl.ds(16 * q, 16)] = e[q] * inv
    return acc


def _sc_router(logits):
    n_rows = logits.shape[0]
    nc, ns = 2, 16
    nw = nc * ns
    rows_per_w = n_rows // nw
    pairs_per_w = rows_per_w // 2
    idx_rows = n_rows // 2

    mesh = plsc.VectorSubcoreMesh(core_axis_name="c", subcore_axis_name="s")

    @functools.partial(
        pl.kernel,
        out_type=[
            jax.ShapeDtypeStruct((n_rows, _NUM_EXPERTS), jnp.float32),
            jax.ShapeDtypeStruct((idx_rows, _LANES), jnp.int32),
        ],
        mesh=mesh,
        scratch_types=[
            pltpu.VMEM((rows_per_w, _NUM_EXPERTS), jnp.float32),
            pltpu.VMEM((rows_per_w, _NUM_EXPERTS), jnp.float32),
            pltpu.VMEM((pairs_per_w, _LANES), jnp.int32),
        ],
    )
    def sc_kernel(logits_hbm, scores_hbm, idx_hbm, logits_v, scores_v, idx_v):
        wid = lax.axis_index("s") * nc + lax.axis_index("c")
        base = wid * rows_per_w
        pltpu.sync_copy(logits_hbm.at[pl.ds(base, rows_per_w)], logits_v)

        lane_iota = lax.iota(jnp.int32, _LANES)
        idx_vecs = [lane_iota + jnp.int32(16 * q) for q in range(4)]
        perms = [jnp.bitwise_xor(lane_iota, jnp.int32(b)) for b in (1, 2, 4, 8)]

        @plsc.parallel_loop(0, pairs_per_w, step=1, unroll=2)
        def pair_body(p):
            acc = jnp.zeros((_LANES,), dtype=jnp.int32)
            acc = _process_row(
                logits_v, scores_v, 2 * p, idx_vecs, lane_iota, perms, 0, acc
            )
            acc = _process_row(
                logits_v, scores_v, 2 * p + 1, idx_vecs, lane_iota, perms, 8, acc
            )
            idx_v[p, ...] = acc

        pltpu.sync_copy(scores_v, scores_hbm.at[pl.ds(base, rows_per_w)])
        pltpu.sync_copy(idx_v, idx_hbm.at[pl.ds(wid * pairs_per_w, pairs_per_w)])

    return sc_kernel(logits)


def kernel(hidden_states, kernel, bias):
    hidden_dim = hidden_states.shape[-1]
    hs = hidden_states.reshape(-1, hidden_dim)
    n_rows = hs.shape[0]
    bias2d = bias.reshape(1, _NUM_EXPERTS)
    logits = _tc_logits(hs, kernel, bias2d)
    scores, idx_packed = _sc_router(logits)
    return scores, idx_packed.reshape(n_rows, _TOP_K)


# speedup vs baseline: 1.5325x; 1.2054x over previous
import functools

import jax
import jax.numpy as jnp
from jax import lax
from jax.experimental import pallas as pl
from jax.experimental.pallas import tpu as pltpu
from jax.experimental.pallas import tpu_sc as plsc

_TOP_K = 8
_NUM_EXPERTS = 64
_ROW_BLOCK = 512
_LANES = 16


def _logits_block(hs_ref, w_ref, b_ref, out_ref):
    out_ref[...] = (
        jnp.dot(hs_ref[...], w_ref[...], preferred_element_type=jnp.float32)
        + b_ref[...]
    )


def _tc_logits(hs, w, bias2d):
    n_rows, hidden_dim = hs.shape
    grid = (n_rows // _ROW_BLOCK,)
    return pl.pallas_call(
        _logits_block,
        grid=grid,
        in_specs=[
            pl.BlockSpec((_ROW_BLOCK, hidden_dim), lambda i: (i, 0)),
            pl.BlockSpec((hidden_dim, _NUM_EXPERTS), lambda i: (0, 0)),
            pl.BlockSpec((1, _NUM_EXPERTS), lambda i: (0, 0)),
        ],
        out_specs=pl.BlockSpec((_ROW_BLOCK, _NUM_EXPERTS), lambda i: (i, 0)),
        out_shape=jax.ShapeDtypeStruct((n_rows, _NUM_EXPERTS), jnp.float32),
        compiler_params=pltpu.CompilerParams(
            dimension_semantics=("arbitrary",),
        ),
    )(hs, w, bias2d)


def _splat_i32(x):
    return jnp.full((_LANES,), x, dtype=jnp.int32)


def _sc_router(logits_flat, n_rows):
    nc, ns = 2, 16
    nw = nc * ns
    rows_per_w = n_rows // nw
    groups_per_w = rows_per_w // _LANES
    scores_per_w = rows_per_w * _NUM_EXPERTS
    idx_per_w = rows_per_w * _TOP_K

    mesh = plsc.VectorSubcoreMesh(core_axis_name="c", subcore_axis_name="s")

    @functools.partial(
        pl.kernel,
        out_type=[
            jax.ShapeDtypeStruct((n_rows * _NUM_EXPERTS,), jnp.float32),
            jax.ShapeDtypeStruct((n_rows * _TOP_K,), jnp.int32),
        ],
        mesh=mesh,
        compiler_params=pltpu.CompilerParams(needs_layout_passes=False),
        scratch_types=[
            pltpu.VMEM((rows_per_w * _NUM_EXPERTS,), jnp.float32),
            pltpu.VMEM((rows_per_w * _NUM_EXPERTS,), jnp.float32),
            pltpu.VMEM((rows_per_w * _TOP_K,), jnp.int32),
        ],
    )
    def sc_kernel(logits_hbm, scores_hbm, idx_hbm, logits_v, scores_v, idx_v):
        wid = lax.axis_index("s") * nc + lax.axis_index("c")
        pltpu.sync_copy(
            logits_hbm.at[pl.ds(wid * scores_per_w, scores_per_w)], logits_v
        )

        lane = lax.iota(jnp.int32, _LANES)
        zeros16 = jnp.zeros((_LANES,), dtype=jnp.float32)
        neg_inf = jnp.float32(-jnp.inf)

        @plsc.parallel_loop(0, groups_per_w, step=1)
        def group_body(g):
            row_base = (g * _LANES + lane) * _NUM_EXPERTS

            val = [jnp.full((_LANES,), neg_inf, dtype=jnp.float32)
                   for _ in range(_TOP_K)]
            idx = [_splat_i32(0) for _ in range(_TOP_K)]
            for e in range(_NUM_EXPERTS):
                v = plsc.load_gather(logits_v, [row_base + e])
                es = _splat_i32(e)
                ge = [v > val[j] for j in range(_TOP_K)]
                new_val = list(val)
                new_idx = list(idx)
                for j in range(_TOP_K - 1, 0, -1):
                    new_val[j] = jnp.where(
                        ge[j], jnp.where(ge[j - 1], val[j - 1], v), val[j]
                    )
                    new_idx[j] = jnp.where(
                        ge[j], jnp.where(ge[j - 1], idx[j - 1], es), idx[j]
                    )
                new_val[0] = jnp.where(ge[0], v, val[0])
                new_idx[0] = jnp.where(ge[0], es, idx[0])
                val, idx = new_val, new_idx

            m = val[0]
            ex = [jnp.exp(val[j] - m) for j in range(_TOP_K)]
            denom = ex[0]
            for j in range(1, _TOP_K):
                denom = denom + ex[j]
            inv = 1.0 / denom

            for r in range(_LANES):
                for q in range(4):
                    scores_v[
                        pl.ds((g * _LANES + r) * _NUM_EXPERTS + 16 * q, 16)
                    ] = zeros16
            idx_base = (g * _LANES + lane) * _TOP_K
            for j in range(_TOP_K):
                plsc.store_scatter(scores_v, [row_base + idx[j]], ex[j] * inv)
                plsc.store_scatter(idx_v, [idx_base + j], idx[j])

        pltpu.sync_copy(
            scores_v, scores_hbm.at[pl.ds(wid * scores_per_w, scores_per_w)]
        )
        pltpu.sync_copy(idx_v, idx_hbm.at[pl.ds(wid * idx_per_w, idx_per_w)])

    return sc_kernel(logits_flat)


def kernel(hidden_states, kernel, bias):
    hidden_dim = hidden_states.shape[-1]
    hs = hidden_states.reshape(-1, hidden_dim)
    n_rows = hs.shape[0]
    bias2d = bias.reshape(1, _NUM_EXPERTS)
    logits = _tc_logits(hs, kernel, bias2d)
    scores_flat, idx_flat = _sc_router(logits.reshape(-1), n_rows)
    return (
        scores_flat.reshape(n_rows, _NUM_EXPERTS),
        idx_flat.reshape(n_rows, _TOP_K),
    )
